# baseline (speedup 1.0000x reference)
import functools

import numpy as np
import jax
import jax.numpy as jnp
from jax import lax
from jax.experimental import pallas as pl
from jax.experimental.pallas import tpu as pltpu
from jax.experimental.pallas import tpu_sc as plsc

BS = 8
LQ = 1024
EMBED = 256
HEADS = 8
HEAD_DIM = 32
LEVELS = 4
POINTS = 4
GRID_H = 32
GRID_W = 32
SUM_PTS = LEVELS * POINTS
SPATIAL = GRID_H * GRID_W
NCOL = HEADS * SUM_PTS
BH = BS * HEADS

_PERM = np.array([h * 2 * SUM_PTS + p * 2 + xy
                  for xy in (0, 1) for h in range(HEADS)
                  for p in range(SUM_PTS)], dtype=np.int32)


def _corners_body(q_ref, rp_ref, wo_ref, bo_ref, wa_ref, ba_ref, iw_ref):
    q = q_ref[0]
    rpt = rp_ref[0]
    dn = (((0,), (1,)), ((), ()))
    off = lax.dot_general(wo_ref[...], q, dn,
                          preferred_element_type=jnp.float32) + bo_ref[...]
    lg = lax.dot_general(wa_ref[...], q, dn,
                         preferred_element_type=jnp.float32) + ba_ref[...]
    parts = []
    for h in range(HEADS):
        s = lg[h * SUM_PTS:(h + 1) * SUM_PTS, :]
        m = jnp.max(s, axis=0, keepdims=True)
        e = jnp.exp(s - m)
        parts.append(e / jnp.sum(e, axis=0, keepdims=True))
    attn = jnp.concatenate(parts, axis=0)

    rx = rpt[0:1, :]
    ry = rpt[1:2, :]
    rw = rpt[2:3, :]
    rh = rpt[3:4, :]
    scale = (1.0 / POINTS) * 0.5
    gx = (rx + off[:NCOL] * (scale * rw)) * GRID_W - 0.5
    gy = (ry + off[NCOL:] * (scale * rh)) * GRID_H - 0.5
    x0 = jnp.floor(gx)
    y0 = jnp.floor(gy)
    wx1 = gx - x0
    wx0 = 1.0 - wx1
    wy1 = gy - y0
    wy0 = 1.0 - wy1
    for ci, (dy, dx) in enumerate(((0, 0), (0, 1), (1, 0), (1, 1))):
        xi = x0 + dx
        yi = y0 + dy
        valid = ((xi >= 0.0) & (xi <= GRID_W - 1.0)
                 & (yi >= 0.0) & (yi <= GRID_H - 1.0))
        wxy = (wx1 if dx else wx0) * (wy1 if dy else wy0) * attn
        w = jnp.where(valid, wxy, 0.0)
        xc = jnp.clip(xi, 0.0, GRID_W - 1.0)
        yc = jnp.clip(yi, 0.0, GRID_H - 1.0)
        idx = (yc * GRID_W + xc).astype(jnp.int32)
        wbits = ((lax.bitcast_convert_type(w, jnp.int32) + 0x8000)
                 & jnp.int32(-65536))
        iw_ref[0, ci] = wbits | idx


def _corners(query, rp_t, wo, bo, wa, ba):
    nb = query.shape[0]
    return pl.pallas_call(
        _corners_body,
        grid=(nb,),
        in_specs=[
            pl.BlockSpec((1, LQ, EMBED), lambda b: (b, 0, 0)),
            pl.BlockSpec((1, 4, LQ), lambda b: (b, 0, 0)),
            pl.BlockSpec((EMBED, 2 * NCOL), lambda b: (0, 0)),
            pl.BlockSpec((2 * NCOL, 1), lambda b: (0, 0)),
            pl.BlockSpec((EMBED, NCOL), lambda b: (0, 0)),
            pl.BlockSpec((NCOL, 1), lambda b: (0, 0)),
        ],
        out_specs=pl.BlockSpec((1, 4, NCOL, LQ), lambda b: (b, 0, 0, 0)),
        out_shape=jax.ShapeDtypeStruct((nb, 4, NCOL, LQ), jnp.int32),
    )(query, rp_t, wo, bo, wa, ba)


def _make_gather_kernel(nbh):
    info = plsc.get_sparse_core_info()
    nc, ns = info.num_cores, info.num_subcores
    nw = nc * ns
    bh_per = nbh // nw
    nqb = LQ // 16
    mesh = plsc.VectorSubcoreMesh(core_axis_name="c", subcore_axis_name="s")

    @functools.partial(
        pl.kernel, mesh=mesh,
        compiler_params=pltpu.CompilerParams(needs_layout_passes=False),
        out_type=jax.ShapeDtypeStruct((nbh, HEAD_DIM * LQ), jnp.float32),
        scratch_types=[
            pltpu.VMEM(((HEAD_DIM // 2) * SPATIAL,), jnp.int32),
            pltpu.VMEM((4, POINTS, LQ), jnp.int32),
            pltpu.VMEM((HEAD_DIM * LQ,), jnp.float32),
        ],
    )
    def gather_kernel(value_hbm, iw_hbm, out_hbm, table_v, iw_v, acc_v):
        wid = lax.axis_index("s") * nc + lax.axis_index("c")

        def bh_body(db, carry0):
            bh = wid * bh_per + db
            b = bh // HEADS
            h = bh - b * HEADS

            def zero_body(qb, carry1):
                zero = jnp.zeros((16,), jnp.float32)
                for c in range(HEAD_DIM):
                    acc_v[pl.ds(c * LQ + qb * 16, 16)] = zero
                return carry1

            lax.fori_loop(0, nqb, zero_body, 0)

            def lvl_body(l, carry2):
                pltpu.sync_copy(value_hbm.at[l, bh], table_v)
                pltpu.sync_copy(iw_hbm.at[b, :, h, l], iw_v)

                def qb_body(qb, carry3):
                    sl = pl.ds(qb * 16, 16)
                    for chalf in range(2):
                        kbase = chalf * (HEAD_DIM // 4)
                        acc = [acc_v[pl.ds((2 * kbase + c) * LQ + qb * 16,
                                           16)]
                               for c in range(HEAD_DIM // 2)]
                        for ci in range(4):
                            for p in range(POINTS):
                                viw = iw_v[ci, p, sl]
                                sidx = viw & jnp.int32(1023)
                                wv = plsc.bitcast(
                                    viw & jnp.int32(-65536), jnp.float32)
                                for k in range(HEAD_DIM // 4):
                                    vi = plsc.load_gather(
                                        table_v,
                                        [sidx + (kbase + k) * SPATIAL])
                                    vb = plsc.bitcast(vi, jnp.bfloat16)
                                    lo, hi = plsc.unpack(
                                        vb,
                                        format=plsc.PackFormat.INTERLEAVED)
                                    acc[2 * k] = acc[2 * k] + wv * lo
                                    acc[2 * k + 1] = acc[2 * k + 1] + wv * hi
                        for c in range(HEAD_DIM // 2):
                            acc_v[pl.ds((2 * kbase + c) * LQ + qb * 16,
                                        16)] = acc[c]
                    return carry3

                lax.fori_loop(0, nqb, qb_body, 0)
                return carry2

            lax.fori_loop(0, LEVELS, lvl_body, 0)
            pltpu.sync_copy(acc_v, out_hbm.at[bh])
            return carry0

        lax.fori_loop(0, bh_per, bh_body, 0)

    return gather_kernel


_GATHER = None

_NSPLIT = 2
_BSH = BS // _NSPLIT
_BHH = _BSH * HEADS


def kernel(query, reference_points, value, value_spatial_shapes,
           W_off, b_off, W_attn, b_attn):
    del value_spatial_shapes
    global _GATHER
    if _GATHER is None:
        _GATHER = _make_gather_kernel(_BHH)

    rp_t = reference_points[:, :, 0, :].transpose(0, 2, 1)
    wo = W_off[:, _PERM]
    bo = b_off[_PERM].reshape(2 * NCOL, 1)
    ba = b_attn.reshape(NCOL, 1)

    vb = lax.bitcast_convert_type(value.astype(jnp.bfloat16),
                                  jnp.uint16).astype(jnp.uint32)
    vb = vb.reshape(LEVELS, BH, HEAD_DIM // 2, 2, SPATIAL)
    vpk = lax.bitcast_convert_type(
        vb[:, :, :, 0, :] | (vb[:, :, :, 1, :] << 16), jnp.int32)
    vpk = vpk.reshape(LEVELS, BH, (HEAD_DIM // 2) * SPATIAL)

    outs = []
    for s in range(_NSPLIT):
        bsl = slice(s * _BSH, (s + 1) * _BSH)
        iw_c = _corners(query[bsl], rp_t[bsl], wo, bo, W_attn, ba)
        iw2 = iw_c.reshape(_BSH, 4, HEADS, LEVELS, POINTS, LQ)
        outs.append(_GATHER(vpk[:, s * _BHH:(s + 1) * _BHH], iw2))
    out = jnp.concatenate(outs, axis=0)
    return (out.reshape(BS, HEADS, HEAD_DIM, LQ)
            .transpose(0, 3, 1, 2)
            .reshape(BS, LQ, HEADS * HEAD_DIM))


# speedup vs baseline: 1.2702x; 1.2702x over previous
import functools

import numpy as np
import jax
import jax.numpy as jnp
from jax import lax
from jax.experimental import pallas as pl
from jax.experimental.pallas import tpu as pltpu
from jax.experimental.pallas import tpu_sc as plsc

BS = 8
LQ = 1024
EMBED = 256
HEADS = 8
HEAD_DIM = 32
LEVELS = 4
POINTS = 4
GRID_H = 32
GRID_W = 32
SUM_PTS = LEVELS * POINTS
SPATIAL = GRID_H * GRID_W
NCOL = HEADS * SUM_PTS
BH = BS * HEADS

_PERM = np.array([h * 2 * SUM_PTS + p * 2 + xy
                  for xy in (0, 1) for h in range(HEADS)
                  for p in range(SUM_PTS)], dtype=np.int32)


def _corners_body(q_ref, rp_ref, wo_ref, bo_ref, wa_ref, ba_ref, iw_ref):
    q = q_ref[0]
    rpt = rp_ref[0]
    dn = (((0,), (1,)), ((), ()))
    off = lax.dot_general(wo_ref[...], q, dn,
                          preferred_element_type=jnp.float32) + bo_ref[...]
    lg = lax.dot_general(wa_ref[...], q, dn,
                         preferred_element_type=jnp.float32) + ba_ref[...]
    parts = []
    for h in range(HEADS):
        s = lg[h * SUM_PTS:(h + 1) * SUM_PTS, :]
        m = jnp.max(s, axis=0, keepdims=True)
        e = jnp.exp(s - m)
        parts.append(e / jnp.sum(e, axis=0, keepdims=True))
    attn = jnp.concatenate(parts, axis=0)

    rx = rpt[0:1, :]
    ry = rpt[1:2, :]
    rw = rpt[2:3, :]
    rh = rpt[3:4, :]
    scale = (1.0 / POINTS) * 0.5
    gx = (rx + off[:NCOL] * (scale * rw)) * GRID_W - 0.5
    gy = (ry + off[NCOL:] * (scale * rh)) * GRID_H - 0.5
    x0 = jnp.floor(gx)
    y0 = jnp.floor(gy)
    wx1 = gx - x0
    wx0 = 1.0 - wx1
    wy1 = gy - y0
    wy0 = 1.0 - wy1
    for ci, (dy, dx) in enumerate(((0, 0), (0, 1), (1, 0), (1, 1))):
        xi = x0 + dx
        yi = y0 + dy
        valid = ((xi >= 0.0) & (xi <= GRID_W - 1.0)
                 & (yi >= 0.0) & (yi <= GRID_H - 1.0))
        wxy = (wx1 if dx else wx0) * (wy1 if dy else wy0) * attn
        w = jnp.where(valid, wxy, 0.0)
        xc = jnp.clip(xi, 0.0, GRID_W - 1.0)
        yc = jnp.clip(yi, 0.0, GRID_H - 1.0)
        idx = (yc * GRID_W + xc).astype(jnp.int32)
        wbits = ((lax.bitcast_convert_type(w, jnp.int32) + 0x8000)
                 & jnp.int32(-65536))
        iw_ref[0, ci] = wbits | idx


def _corners(query, rp_t, wo, bo, wa, ba):
    nb = query.shape[0]
    return pl.pallas_call(
        _corners_body,
        grid=(nb,),
        in_specs=[
            pl.BlockSpec((1, LQ, EMBED), lambda b: (b, 0, 0)),
            pl.BlockSpec((1, 4, LQ), lambda b: (b, 0, 0)),
            pl.BlockSpec((EMBED, 2 * NCOL), lambda b: (0, 0)),
            pl.BlockSpec((2 * NCOL, 1), lambda b: (0, 0)),
            pl.BlockSpec((EMBED, NCOL), lambda b: (0, 0)),
            pl.BlockSpec((NCOL, 1), lambda b: (0, 0)),
        ],
        out_specs=pl.BlockSpec((1, 4, NCOL, LQ), lambda b: (b, 0, 0, 0)),
        out_shape=jax.ShapeDtypeStruct((nb, 4, NCOL, LQ), jnp.int32),
    )(query, rp_t, wo, bo, wa, ba)


def _pack_body(v_ref, out_ref):
    v = v_ref[0].astype(jnp.bfloat16)
    lo = lax.bitcast_convert_type(v[:, :HEAD_DIM // 2, :],
                                  jnp.uint16).astype(jnp.uint32)
    hi = lax.bitcast_convert_type(v[:, HEAD_DIM // 2:, :],
                                  jnp.uint16).astype(jnp.uint32)
    out_ref[0] = lax.bitcast_convert_type(lo | (hi << 16), jnp.int32)


def _pack_value(value):
    return pl.pallas_call(
        _pack_body,
        grid=(LEVELS,),
        in_specs=[pl.BlockSpec((1, BH, HEAD_DIM, SPATIAL),
                               lambda l: (l, 0, 0, 0))],
        out_specs=pl.BlockSpec((1, BH, HEAD_DIM // 2, SPATIAL),
                               lambda l: (l, 0, 0, 0)),
        out_shape=jax.ShapeDtypeStruct((LEVELS, BH, HEAD_DIM // 2, SPATIAL),
                                       jnp.int32),
    )(value)


def _make_gather_kernel(nbh):
    info = plsc.get_sparse_core_info()
    nc, ns = info.num_cores, info.num_subcores
    nw = nc * ns
    bh_per = nbh // nw
    nqb = LQ // 16
    mesh = plsc.VectorSubcoreMesh(core_axis_name="c", subcore_axis_name="s")

    @functools.partial(
        pl.kernel, mesh=mesh,
        compiler_params=pltpu.CompilerParams(needs_layout_passes=False),
        out_type=jax.ShapeDtypeStruct((nbh, HEAD_DIM * LQ), jnp.float32),
        scratch_types=[
            pltpu.VMEM(((HEAD_DIM // 2) * SPATIAL,), jnp.int32),
            pltpu.VMEM((4, POINTS, LQ), jnp.int32),
            pltpu.VMEM((HEAD_DIM * LQ,), jnp.float32),
        ],
    )
    def gather_kernel(value_hbm, iw_hbm, out_hbm, table_v, iw_v, acc_v):
        wid = lax.axis_index("s") * nc + lax.axis_index("c")

        def bh_body(db, carry0):
            bh = wid * bh_per + db
            b = bh // HEADS
            h = bh - b * HEADS

            def zero_body(qb, carry1):
                zero = jnp.zeros((16,), jnp.float32)
                for c in range(HEAD_DIM):
                    acc_v[pl.ds(c * LQ + qb * 16, 16)] = zero
                return carry1

            lax.fori_loop(0, nqb, zero_body, 0)

            def lvl_body(l, carry2):
                pltpu.sync_copy(value_hbm.at[l, bh], table_v)
                pltpu.sync_copy(iw_hbm.at[b, :, h, l], iw_v)

                def qb_body(qb, carry3):
                    sl = pl.ds(qb * 16, 16)
                    nk = HEAD_DIM // 4
                    for chalf in range(2):
                        kbase = chalf * nk
                        acc_lo = [acc_v[pl.ds((kbase + k) * LQ + qb * 16, 16)]
                                  for k in range(nk)]
                        acc_hi = [acc_v[pl.ds(
                            (kbase + k + HEAD_DIM // 2) * LQ + qb * 16, 16)]
                            for k in range(nk)]
                        for ci in range(4):
                            for p in range(POINTS):
                                viw = iw_v[ci, p, sl]
                                sidx = viw & jnp.int32(1023)
                                wv = plsc.bitcast(
                                    viw & jnp.int32(-65536), jnp.float32)
                                for k in range(nk):
                                    vi = plsc.load_gather(
                                        table_v,
                                        [sidx + (kbase + k) * SPATIAL])
                                    vb = plsc.bitcast(vi, jnp.bfloat16)
                                    lo, hi = plsc.unpack(
                                        vb,
                                        format=plsc.PackFormat.INTERLEAVED)
                                    acc_lo[k] = acc_lo[k] + wv * lo
                                    acc_hi[k] = acc_hi[k] + wv * hi
                        for k in range(nk):
                            acc_v[pl.ds((kbase + k) * LQ + qb * 16, 16)] = (
                                acc_lo[k])
                            acc_v[pl.ds(
                                (kbase + k + HEAD_DIM // 2) * LQ + qb * 16,
                                16)] = acc_hi[k]
                    return carry3

                lax.fori_loop(0, nqb, qb_body, 0)
                return carry2

            lax.fori_loop(0, LEVELS, lvl_body, 0)
            pltpu.sync_copy(acc_v, out_hbm.at[bh])
            return carry0

        lax.fori_loop(0, bh_per, bh_body, 0)

    return gather_kernel


_GATHER = None

_NSPLIT = 2
_BSH = BS // _NSPLIT
_BHH = _BSH * HEADS


def kernel(query, reference_points, value, value_spatial_shapes,
           W_off, b_off, W_attn, b_attn):
    del value_spatial_shapes
    global _GATHER
    if _GATHER is None:
        _GATHER = _make_gather_kernel(_BHH)

    rp_t = reference_points[:, :, 0, :].transpose(0, 2, 1)
    wo = W_off[:, _PERM]
    bo = b_off[_PERM].reshape(2 * NCOL, 1)
    ba = b_attn.reshape(NCOL, 1)

    vpk = _pack_value(value.reshape(LEVELS, BH, HEAD_DIM, SPATIAL))
    vpk = vpk.reshape(LEVELS, BH, (HEAD_DIM // 2) * SPATIAL)

    outs = []
    for s in range(_NSPLIT):
        bsl = slice(s * _BSH, (s + 1) * _BSH)
        iw_c = _corners(query[bsl], rp_t[bsl], wo, bo, W_attn, ba)
        iw2 = iw_c.reshape(_BSH, 4, HEADS, LEVELS, POINTS, LQ)
        out_s = _GATHER(vpk[:, s * _BHH:(s + 1) * _BHH], iw2)
        outs.append(out_s.reshape(_BSH, HEADS, HEAD_DIM, LQ)
                    .transpose(0, 3, 1, 2)
                    .reshape(_BSH, LQ, HEADS * HEAD_DIM))
    return jnp.concatenate(outs, axis=0)
